# Initial kernel scaffold; baseline (speedup 1.0000x reference)
import jax
import jax.numpy as jnp
from jax.experimental import pallas as pl


def kernel(x, p_pos, W_in, W_out):
    raise NotImplementedError("write your pallas kernel here")


# speedup vs baseline: 1.0058x; 1.0058x over previous
import functools

import jax
import jax.numpy as jnp
from jax.experimental import pallas as pl

_STRIDE, _K = 4, 16


def _fps_one(pos, n_out):
    Np = pos.shape[0]
    dists = jnp.full((Np,), jnp.inf, dtype=pos.dtype)
    idxs = jnp.zeros((n_out,), dtype=jnp.int32)

    def body(i, carry):
        dists, idxs = carry
        cur = pos[idxs[i - 1]]
        d = jnp.sum((pos - cur[None, :]) ** 2, axis=-1)
        dists = jnp.minimum(dists, d)
        nxt = jnp.argmax(dists).astype(jnp.int32)
        idxs = idxs.at[i].set(nxt)
        return (dists, idxs)

    _, idxs = jax.lax.fori_loop(1, n_out, body, (dists, idxs))
    return idxs


def _mm_kernel(a_ref, b_ref, o_ref):
    o_ref[...] = jnp.dot(a_ref[...], b_ref[...],
                         preferred_element_type=jnp.float32)


def _matmul(a, b, bm=512):
    M, Kd = a.shape
    _, N = b.shape
    return pl.pallas_call(
        _mm_kernel,
        grid=(M // bm,),
        in_specs=[pl.BlockSpec((bm, Kd), lambda i: (i, 0)),
                  pl.BlockSpec((Kd, N), lambda i: (0, 0))],
        out_specs=pl.BlockSpec((bm, N), lambda i: (i, 0)),
        out_shape=jax.ShapeDtypeStruct((M, N), jnp.float32),
    )(a, b)


def _head_kernel(g_ref, pw_ref, w_ref, o_ref):
    h = g_ref[...] + pw_ref[...]
    h = jnp.where(h > 0, h, 0.01 * h)
    o_ref[...] = jnp.dot(h, w_ref[...], preferred_element_type=jnp.float32)


def _head(gmax, posw, w_out, bm=512):
    M, D = gmax.shape
    return pl.pallas_call(
        _head_kernel,
        grid=(M // bm,),
        in_specs=[pl.BlockSpec((bm, D), lambda i: (i, 0)),
                  pl.BlockSpec((bm, D), lambda i: (i, 0)),
                  pl.BlockSpec((D, D), lambda i: (0, 0))],
        out_specs=pl.BlockSpec((bm, D), lambda i: (i, 0)),
        out_shape=jax.ShapeDtypeStruct((M, D), jnp.float32),
    )(gmax, posw, w_out)


def kernel(x, p_pos, W_in, W_out):
    b, n, d_in = x.shape
    n_out = n // _STRIDE
    d_out = W_in.shape[1]

    fps_idx = jax.vmap(lambda p: _fps_one(p, n_out))(p_pos)
    fp_p_pos = jnp.take_along_axis(p_pos, fps_idx[..., None], axis=1)

    d2 = (jnp.sum(fp_p_pos ** 2, axis=-1, keepdims=True)
          - 2.0 * jnp.einsum('bqd,bnd->bqn', fp_p_pos, p_pos)
          + jnp.sum(p_pos ** 2, axis=-1)[:, None, :])
    _, knn_idx = jax.lax.top_k(-d2, _K)

    xW = _matmul(x.reshape(b * n, d_in), W_in[3:]).reshape(b, n, d_out)
    g = jax.vmap(lambda t, i: t[i])(xW, knn_idx)
    gmax = jnp.max(g, axis=-2)

    posw = fp_p_pos @ W_in[:3]
    h = _head(gmax.reshape(b * n_out, d_out),
              posw.reshape(b * n_out, d_out), W_out)
    return (h.reshape(b, n_out, d_out), fp_p_pos)


# speedup vs baseline: 2.3404x; 2.3268x over previous
import functools

import jax
import jax.numpy as jnp
from jax.experimental import pallas as pl

_STRIDE, _K = 4, 16


def _fps_kernel(px_ref, py_ref, pz_ref, idx_ref, fx_ref, fy_ref, fz_ref,
                *, n_out, rows):
    px = px_ref[0]
    py = py_ref[0]
    pz = pz_ref[0]
    lin = (jax.lax.broadcasted_iota(jnp.int32, (rows, 128), 0) * 128
           + jax.lax.broadcasted_iota(jnp.int32, (rows, 128), 1))
    orows = n_out // 128
    lin_o = (jax.lax.broadcasted_iota(jnp.int32, (orows, 128), 0) * 128
             + jax.lax.broadcasted_iota(jnp.int32, (orows, 128), 1))
    inf = jnp.float32(jnp.inf)
    big = jnp.int32(2147483647)

    def extract(ci):
        msk = (lin == ci)
        cx = jnp.sum(jnp.where(msk, px, 0.0))
        cy = jnp.sum(jnp.where(msk, py, 0.0))
        cz = jnp.sum(jnp.where(msk, pz, 0.0))
        return cx, cy, cz

    def body(i, carry):
        dists, cur, acc_i, acc_x, acc_y, acc_z = carry
        cx, cy, cz = extract(cur)
        dx = px - cx
        dy = py - cy
        dz = pz - cz
        d = (dx * dx + dy * dy) + dz * dz
        dists = jnp.minimum(dists, d)
        m = jnp.max(dists)
        nxt = jnp.min(jnp.where(dists == m, lin, big))
        slot_prev = (lin_o == (i - 1))
        slot_cur = (lin_o == i)
        acc_x = jnp.where(slot_prev, cx, acc_x)
        acc_y = jnp.where(slot_prev, cy, acc_y)
        acc_z = jnp.where(slot_prev, cz, acc_z)
        acc_i = jnp.where(slot_cur, nxt, acc_i)
        return (dists, nxt, acc_i, acc_x, acc_y, acc_z)

    init = (jnp.full((rows, 128), inf, dtype=jnp.float32),
            jnp.int32(0),
            jnp.zeros((orows, 128), dtype=jnp.int32),
            jnp.zeros((orows, 128), dtype=jnp.float32),
            jnp.zeros((orows, 128), dtype=jnp.float32),
            jnp.zeros((orows, 128), dtype=jnp.float32))
    _, last, acc_i, acc_x, acc_y, acc_z = jax.lax.fori_loop(
        1, n_out, body, init)
    cx, cy, cz = extract(last)
    slot_last = (lin_o == (n_out - 1))
    idx_ref[0] = acc_i
    fx_ref[0] = jnp.where(slot_last, cx, acc_x)
    fy_ref[0] = jnp.where(slot_last, cy, acc_y)
    fz_ref[0] = jnp.where(slot_last, cz, acc_z)


def _fps(p_pos, n_out):
    b, n, _ = p_pos.shape
    rows = n // 128
    orows = n_out // 128
    px = p_pos[..., 0].reshape(b, rows, 128)
    py = p_pos[..., 1].reshape(b, rows, 128)
    pz = p_pos[..., 2].reshape(b, rows, 128)
    spec_in = pl.BlockSpec((1, rows, 128), lambda i: (i, 0, 0))
    spec_out = pl.BlockSpec((1, orows, 128), lambda i: (i, 0, 0))
    out_shape = [jax.ShapeDtypeStruct((b, orows, 128), jnp.int32)] + \
        [jax.ShapeDtypeStruct((b, orows, 128), jnp.float32)] * 3
    idx, fx, fy, fz = pl.pallas_call(
        functools.partial(_fps_kernel, n_out=n_out, rows=rows),
        grid=(b,),
        in_specs=[spec_in] * 3,
        out_specs=[spec_out] * 4,
        out_shape=out_shape,
    )(px, py, pz)
    fp = jnp.stack([fx.reshape(b, n_out), fy.reshape(b, n_out),
                    fz.reshape(b, n_out)], axis=-1)
    return idx.reshape(b, n_out), fp


def _mm_kernel(a_ref, b_ref, o_ref):
    o_ref[...] = jnp.dot(a_ref[...], b_ref[...],
                         preferred_element_type=jnp.float32)


def _matmul(a, b, bm=512):
    M, Kd = a.shape
    _, N = b.shape
    return pl.pallas_call(
        _mm_kernel,
        grid=(M // bm,),
        in_specs=[pl.BlockSpec((bm, Kd), lambda i: (i, 0)),
                  pl.BlockSpec((Kd, N), lambda i: (0, 0))],
        out_specs=pl.BlockSpec((bm, N), lambda i: (i, 0)),
        out_shape=jax.ShapeDtypeStruct((M, N), jnp.float32),
    )(a, b)


def _head_kernel(g_ref, pw_ref, w_ref, o_ref):
    h = g_ref[...] + pw_ref[...]
    h = jnp.where(h > 0, h, 0.01 * h)
    o_ref[...] = jnp.dot(h, w_ref[...], preferred_element_type=jnp.float32)


def _head(gmax, posw, w_out, bm=512):
    M, D = gmax.shape
    return pl.pallas_call(
        _head_kernel,
        grid=(M // bm,),
        in_specs=[pl.BlockSpec((bm, D), lambda i: (i, 0)),
                  pl.BlockSpec((bm, D), lambda i: (i, 0)),
                  pl.BlockSpec((D, D), lambda i: (0, 0))],
        out_specs=pl.BlockSpec((bm, D), lambda i: (i, 0)),
        out_shape=jax.ShapeDtypeStruct((M, D), jnp.float32),
    )(gmax, posw, w_out)


def kernel(x, p_pos, W_in, W_out):
    b, n, d_in = x.shape
    n_out = n // _STRIDE
    d_out = W_in.shape[1]

    fps_idx, fp_p_pos = _fps(p_pos, n_out)

    d2 = (jnp.sum(fp_p_pos ** 2, axis=-1, keepdims=True)
          - 2.0 * jnp.einsum('bqd,bnd->bqn', fp_p_pos, p_pos)
          + jnp.sum(p_pos ** 2, axis=-1)[:, None, :])
    _, knn_idx = jax.lax.top_k(-d2, _K)

    xW = _matmul(x.reshape(b * n, d_in), W_in[3:]).reshape(b, n, d_out)
    g = jax.vmap(lambda t, i: t[i])(xW, knn_idx)
    gmax = jnp.max(g, axis=-2)

    posw = fp_p_pos @ W_in[:3]
    h = _head(gmax.reshape(b * n_out, d_out),
              posw.reshape(b * n_out, d_out), W_out)
    return (h.reshape(b, n_out, d_out), fp_p_pos)


# speedup vs baseline: 6.9574x; 2.9728x over previous
import functools

import jax
import jax.numpy as jnp
from jax.experimental import pallas as pl
from jax.experimental.pallas import tpu as pltpu

_STRIDE, _K = 4, 16


def _fps_kernel(px_ref, py_ref, pz_ref, idx_ref, fx_ref, fy_ref, fz_ref,
                *, n_out, rows):
    px = px_ref[0]
    py = py_ref[0]
    pz = pz_ref[0]
    lin = (jax.lax.broadcasted_iota(jnp.int32, (rows, 128), 0) * 128
           + jax.lax.broadcasted_iota(jnp.int32, (rows, 128), 1))
    orows = n_out // 128
    lin_o = (jax.lax.broadcasted_iota(jnp.int32, (orows, 128), 0) * 128
             + jax.lax.broadcasted_iota(jnp.int32, (orows, 128), 1))
    inf = jnp.float32(jnp.inf)
    big = jnp.int32(2147483647)

    def extract(ci):
        msk = (lin == ci)
        cx = jnp.sum(jnp.where(msk, px, 0.0))
        cy = jnp.sum(jnp.where(msk, py, 0.0))
        cz = jnp.sum(jnp.where(msk, pz, 0.0))
        return cx, cy, cz

    def body(i, carry):
        dists, cur, acc_i, acc_x, acc_y, acc_z = carry
        cx, cy, cz = extract(cur)
        dx = px - cx
        dy = py - cy
        dz = pz - cz
        d = (dx * dx + dy * dy) + dz * dz
        dists = jnp.minimum(dists, d)
        m = jnp.max(dists)
        nxt = jnp.min(jnp.where(dists == m, lin, big))
        slot_prev = (lin_o == (i - 1))
        slot_cur = (lin_o == i)
        acc_x = jnp.where(slot_prev, cx, acc_x)
        acc_y = jnp.where(slot_prev, cy, acc_y)
        acc_z = jnp.where(slot_prev, cz, acc_z)
        acc_i = jnp.where(slot_cur, nxt, acc_i)
        return (dists, nxt, acc_i, acc_x, acc_y, acc_z)

    init = (jnp.full((rows, 128), inf, dtype=jnp.float32),
            jnp.int32(0),
            jnp.zeros((orows, 128), dtype=jnp.int32),
            jnp.zeros((orows, 128), dtype=jnp.float32),
            jnp.zeros((orows, 128), dtype=jnp.float32),
            jnp.zeros((orows, 128), dtype=jnp.float32))
    _, last, acc_i, acc_x, acc_y, acc_z = jax.lax.fori_loop(
        1, n_out, body, init)
    cx, cy, cz = extract(last)
    slot_last = (lin_o == (n_out - 1))
    idx_ref[0] = acc_i
    fx_ref[0] = jnp.where(slot_last, cx, acc_x)
    fy_ref[0] = jnp.where(slot_last, cy, acc_y)
    fz_ref[0] = jnp.where(slot_last, cz, acc_z)


def _fps(p_pos, n_out):
    b, n, _ = p_pos.shape
    rows = n // 128
    orows = n_out // 128
    px = p_pos[..., 0].reshape(b, rows, 128)
    py = p_pos[..., 1].reshape(b, rows, 128)
    pz = p_pos[..., 2].reshape(b, rows, 128)
    spec_in = pl.BlockSpec((1, rows, 128), lambda i: (i, 0, 0))
    spec_out = pl.BlockSpec((1, orows, 128), lambda i: (i, 0, 0))
    out_shape = [jax.ShapeDtypeStruct((b, orows, 128), jnp.int32)] + \
        [jax.ShapeDtypeStruct((b, orows, 128), jnp.float32)] * 3
    idx, fx, fy, fz = pl.pallas_call(
        functools.partial(_fps_kernel, n_out=n_out, rows=rows),
        grid=(b,),
        in_specs=[spec_in] * 3,
        out_specs=[spec_out] * 4,
        out_shape=out_shape,
    )(px, py, pz)
    fp = jnp.stack([fx.reshape(b, n_out), fy.reshape(b, n_out),
                    fz.reshape(b, n_out)], axis=-1)
    return idx.reshape(b, n_out), fp


def _knn_ab_kernel(q_ref, p_ref, d2_ref, seg_ref, m_ref, *, n, nseg):
    q8 = q_ref[...]
    q2 = jnp.sum(q8 * q8, axis=1, keepdims=True)
    ch = 1024
    for c in range(n // ch):
        sl = slice(c * ch, (c + 1) * ch)
        p8c = p_ref[0, :, sl]
        dt = jnp.dot(q8, p8c, preferred_element_type=jnp.float32)
        p2 = jnp.sum(p8c * p8c, axis=0, keepdims=True)
        d2c = (q2 - 2.0 * dt) + p2
        d2_ref[:, sl] = d2c
        m_ref[:, c * (ch // 16):(c + 1) * (ch // 16)] = jnp.min(
            d2c.reshape(128, ch // 16, 16), axis=2)

    segio = jax.lax.broadcasted_iota(jnp.int32, (128, nseg), 1)
    colio = jax.lax.broadcasted_iota(jnp.int32, (128, 16), 1)
    acc = jnp.zeros((128, 16), dtype=jnp.int32)
    big = jnp.int32(2147483647)
    inf = jnp.float32(jnp.inf)
    for t in range(16):
        M = m_ref[...]
        m = jnp.min(M, axis=1, keepdims=True)
        sid = jnp.min(jnp.where(M == m, segio, big), axis=1, keepdims=True)
        m_ref[...] = jnp.where((M == m) & (segio == sid), inf, M)
        acc = jnp.where(colio == t, sid, acc)
    seg_ref[...] = acc


def _knn_ab(fp_pos, p_pos):
    b, n_out, _ = fp_pos.shape
    n = p_pos.shape[1]
    nq = b * n_out
    nseg = n // 16
    q8 = jnp.pad(fp_pos.reshape(nq, 3), ((0, 0), (0, 5)))
    p8t = jnp.pad(jnp.swapaxes(p_pos, 1, 2), ((0, 0), (0, 5), (0, 0)))
    blocks_per_b = n_out // 128
    d2, seg16 = pl.pallas_call(
        functools.partial(_knn_ab_kernel, n=n, nseg=nseg),
        grid=(nq // 128,),
        in_specs=[
            pl.BlockSpec((128, 8), lambda i: (i, 0)),
            pl.BlockSpec((1, 8, n), lambda i: (i // blocks_per_b, 0, 0)),
        ],
        out_specs=[
            pl.BlockSpec((128, n), lambda i: (i, 0)),
            pl.BlockSpec((128, 16), lambda i: (i, 0)),
        ],
        out_shape=[
            jax.ShapeDtypeStruct((nq, n), jnp.float32),
            jax.ShapeDtypeStruct((nq, 16), jnp.int32),
        ],
        scratch_shapes=[pltpu.VMEM((128, nseg), jnp.float32)],
    )(q8, p8t)
    return d2, seg16


def _mm_kernel(a_ref, b_ref, o_ref):
    o_ref[...] = jnp.dot(a_ref[...], b_ref[...],
                         preferred_element_type=jnp.float32)


def _matmul(a, b, bm=512):
    M, Kd = a.shape
    _, N = b.shape
    return pl.pallas_call(
        _mm_kernel,
        grid=(M // bm,),
        in_specs=[pl.BlockSpec((bm, Kd), lambda i: (i, 0)),
                  pl.BlockSpec((Kd, N), lambda i: (0, 0))],
        out_specs=pl.BlockSpec((bm, N), lambda i: (i, 0)),
        out_shape=jax.ShapeDtypeStruct((M, N), jnp.float32),
    )(a, b)


def _head_kernel(g_ref, pw_ref, w_ref, o_ref):
    h = g_ref[...] + pw_ref[...]
    h = jnp.where(h > 0, h, 0.01 * h)
    o_ref[...] = jnp.dot(h, w_ref[...], preferred_element_type=jnp.float32)


def _head(gmax, posw, w_out, bm=512):
    M, D = gmax.shape
    return pl.pallas_call(
        _head_kernel,
        grid=(M // bm,),
        in_specs=[pl.BlockSpec((bm, D), lambda i: (i, 0)),
                  pl.BlockSpec((bm, D), lambda i: (i, 0)),
                  pl.BlockSpec((D, D), lambda i: (0, 0))],
        out_specs=pl.BlockSpec((bm, D), lambda i: (i, 0)),
        out_shape=jax.ShapeDtypeStruct((M, D), jnp.float32),
    )(gmax, posw, w_out)


def kernel(x, p_pos, W_in, W_out):
    b, n, d_in = x.shape
    n_out = n // _STRIDE
    d_out = W_in.shape[1]

    fps_idx, fp_p_pos = _fps(p_pos, n_out)

    d2, seg16 = _knn_ab(fp_p_pos, p_pos)
    seg16 = jnp.sort(seg16, axis=-1)
    colidx = (seg16[..., None] * 16
              + jnp.arange(16, dtype=jnp.int32)).reshape(b * n_out, 256)
    cand = jnp.take_along_axis(d2, colidx, axis=1)
    _, pos256 = jax.lax.top_k(-cand, _K)
    knn_idx = jnp.take_along_axis(colidx, pos256, axis=1).reshape(
        b, n_out, _K)

    xW = _matmul(x.reshape(b * n, d_in), W_in[3:]).reshape(b, n, d_out)
    g = jax.vmap(lambda t, i: t[i])(xW, knn_idx)
    gmax = jnp.max(g, axis=-2)

    posw = fp_p_pos @ W_in[:3]
    h = _head(gmax.reshape(b * n_out, d_out),
              posw.reshape(b * n_out, d_out), W_out)
    return (h.reshape(b, n_out, d_out), fp_p_pos)


# speedup vs baseline: 8.5146x; 1.2238x over previous
import functools

import jax
import jax.numpy as jnp
from jax.experimental import pallas as pl
from jax.experimental.pallas import tpu as pltpu

_STRIDE, _K = 4, 16


def _fps_kernel(px_ref, py_ref, pz_ref, sx_ref, sy_ref, sz_ref,
                idx_ref, fx_ref, fy_ref, fz_ref,
                *, n_out, rows, nb):
    lin = (jax.lax.broadcasted_iota(jnp.int32, (rows, 128), 0) * 128
           + jax.lax.broadcasted_iota(jnp.int32, (rows, 128), 1))
    orows = n_out // 128
    lin_o = (jax.lax.broadcasted_iota(jnp.int32, (orows, 128), 0) * 128
             + jax.lax.broadcasted_iota(jnp.int32, (orows, 128), 1))
    inf = jnp.float32(jnp.inf)
    big = jnp.int32(2147483647)
    p = [(px_ref[b1], py_ref[b1], pz_ref[b1]) for b1 in range(nb)]

    def step(b1, i, dists, cur, accs):
        acc_i, acc_x, acc_y, acc_z = accs
        cx = sx_ref[b1, cur]
        cy = sy_ref[b1, cur]
        cz = sz_ref[b1, cur]
        px, py, pz = p[b1]
        dx = px - cx
        dy = py - cy
        dz = pz - cz
        d = (dx * dx + dy * dy) + dz * dz
        dists = jnp.minimum(dists, d)
        m = jnp.max(dists)
        nxt = jnp.min(jnp.where(dists == m, lin, big))
        slot_prev = (lin_o == (i - 1))
        slot_cur = (lin_o == i)
        acc_x = jnp.where(slot_prev, cx, acc_x)
        acc_y = jnp.where(slot_prev, cy, acc_y)
        acc_z = jnp.where(slot_prev, cz, acc_z)
        acc_i = jnp.where(slot_cur, nxt, acc_i)
        return dists, nxt, (acc_i, acc_x, acc_y, acc_z)

    def body(i, carry):
        out = []
        for b1 in range(nb):
            dists, cur, accs = carry[b1]
            out.append(step(b1, i, dists, cur, accs))
        return tuple(out)

    def init_one():
        return (jnp.full((rows, 128), inf, dtype=jnp.float32),
                jnp.int32(0),
                (jnp.zeros((orows, 128), dtype=jnp.int32),
                 jnp.zeros((orows, 128), dtype=jnp.float32),
                 jnp.zeros((orows, 128), dtype=jnp.float32),
                 jnp.zeros((orows, 128), dtype=jnp.float32)))

    carry = jax.lax.fori_loop(1, n_out, body,
                              tuple(init_one() for _ in range(nb)))
    slot_last = (lin_o == (n_out - 1))
    for b1 in range(nb):
        _, last, (acc_i, acc_x, acc_y, acc_z) = carry[b1]
        idx_ref[b1] = acc_i
        fx_ref[b1] = jnp.where(slot_last, sx_ref[b1, last], acc_x)
        fy_ref[b1] = jnp.where(slot_last, sy_ref[b1, last], acc_y)
        fz_ref[b1] = jnp.where(slot_last, sz_ref[b1, last], acc_z)


def _fps(p_pos, n_out):
    b, n, _ = p_pos.shape
    rows = n // 128
    orows = n_out // 128
    px = p_pos[..., 0].reshape(b, rows, 128)
    py = p_pos[..., 1].reshape(b, rows, 128)
    pz = p_pos[..., 2].reshape(b, rows, 128)
    sx = p_pos[..., 0]
    sy = p_pos[..., 1]
    sz = p_pos[..., 2]
    spec_v = pl.BlockSpec((b, rows, 128), lambda: (0, 0, 0))
    spec_s = pl.BlockSpec(memory_space=pltpu.SMEM)
    spec_out = pl.BlockSpec((b, orows, 128), lambda: (0, 0, 0))
    out_shape = [jax.ShapeDtypeStruct((b, orows, 128), jnp.int32)] + \
        [jax.ShapeDtypeStruct((b, orows, 128), jnp.float32)] * 3
    idx, fx, fy, fz = pl.pallas_call(
        functools.partial(_fps_kernel, n_out=n_out, rows=rows, nb=b),
        in_specs=[spec_v] * 3 + [spec_s] * 3,
        out_specs=[spec_out] * 4,
        out_shape=out_shape,
    )(px, py, pz, sx, sy, sz)
    fp = jnp.stack([fx.reshape(b, n_out), fy.reshape(b, n_out),
                    fz.reshape(b, n_out)], axis=-1)
    return idx.reshape(b, n_out), fp


def _knn_ab_kernel(q_ref, p_ref, d2_ref, seg_ref, m_ref, *, n, nseg):
    q8 = q_ref[...]
    q2 = jnp.sum(q8 * q8, axis=1, keepdims=True)
    ch = 1024
    for c in range(n // ch):
        sl = slice(c * ch, (c + 1) * ch)
        p8c = p_ref[0, :, sl]
        dt = jnp.dot(q8, p8c, preferred_element_type=jnp.float32)
        p2 = jnp.sum(p8c * p8c, axis=0, keepdims=True)
        d2c = (q2 - 2.0 * dt) + p2
        d2_ref[:, sl] = d2c
        m_ref[:, c * (ch // 16):(c + 1) * (ch // 16)] = jnp.min(
            d2c.reshape(128, ch // 16, 16), axis=2)

    segio = jax.lax.broadcasted_iota(jnp.int32, (128, nseg), 1)
    colio = jax.lax.broadcasted_iota(jnp.int32, (128, 16), 1)
    acc = jnp.zeros((128, 16), dtype=jnp.int32)
    big = jnp.int32(2147483647)
    inf = jnp.float32(jnp.inf)
    for t in range(16):
        M = m_ref[...]
        m = jnp.min(M, axis=1, keepdims=True)
        sid = jnp.min(jnp.where(M == m, segio, big), axis=1, keepdims=True)
        m_ref[...] = jnp.where((M == m) & (segio == sid), inf, M)
        acc = jnp.where(colio == t, sid, acc)
    seg_ref[...] = acc


def _knn_ab(fp_pos, p_pos):
    b, n_out, _ = fp_pos.shape
    n = p_pos.shape[1]
    nq = b * n_out
    nseg = n // 16
    q8 = jnp.pad(fp_pos.reshape(nq, 3), ((0, 0), (0, 5)))
    p8t = jnp.pad(jnp.swapaxes(p_pos, 1, 2), ((0, 0), (0, 5), (0, 0)))
    blocks_per_b = n_out // 128
    d2, seg16 = pl.pallas_call(
        functools.partial(_knn_ab_kernel, n=n, nseg=nseg),
        grid=(nq // 128,),
        in_specs=[
            pl.BlockSpec((128, 8), lambda i: (i, 0)),
            pl.BlockSpec((1, 8, n), lambda i: (i // blocks_per_b, 0, 0)),
        ],
        out_specs=[
            pl.BlockSpec((128, n), lambda i: (i, 0)),
            pl.BlockSpec((128, 16), lambda i: (i, 0)),
        ],
        out_shape=[
            jax.ShapeDtypeStruct((nq, n), jnp.float32),
            jax.ShapeDtypeStruct((nq, 16), jnp.int32),
        ],
        scratch_shapes=[pltpu.VMEM((128, nseg), jnp.float32)],
    )(q8, p8t)
    return d2, seg16


def _mm_kernel(a_ref, b_ref, o_ref):
    o_ref[...] = jnp.dot(a_ref[...], b_ref[...],
                         preferred_element_type=jnp.float32)


def _matmul(a, b, bm=512):
    M, Kd = a.shape
    _, N = b.shape
    return pl.pallas_call(
        _mm_kernel,
        grid=(M // bm,),
        in_specs=[pl.BlockSpec((bm, Kd), lambda i: (i, 0)),
                  pl.BlockSpec((Kd, N), lambda i: (0, 0))],
        out_specs=pl.BlockSpec((bm, N), lambda i: (i, 0)),
        out_shape=jax.ShapeDtypeStruct((M, N), jnp.float32),
    )(a, b)


def _head_kernel(g_ref, pw_ref, w_ref, o_ref):
    h = g_ref[...] + pw_ref[...]
    h = jnp.where(h > 0, h, 0.01 * h)
    o_ref[...] = jnp.dot(h, w_ref[...], preferred_element_type=jnp.float32)


def _head(gmax, posw, w_out, bm=512):
    M, D = gmax.shape
    return pl.pallas_call(
        _head_kernel,
        grid=(M // bm,),
        in_specs=[pl.BlockSpec((bm, D), lambda i: (i, 0)),
                  pl.BlockSpec((bm, D), lambda i: (i, 0)),
                  pl.BlockSpec((D, D), lambda i: (0, 0))],
        out_specs=pl.BlockSpec((bm, D), lambda i: (i, 0)),
        out_shape=jax.ShapeDtypeStruct((M, D), jnp.float32),
    )(gmax, posw, w_out)


def kernel(x, p_pos, W_in, W_out):
    b, n, d_in = x.shape
    n_out = n // _STRIDE
    d_out = W_in.shape[1]

    fps_idx, fp_p_pos = _fps(p_pos, n_out)

    d2, seg16 = _knn_ab(fp_p_pos, p_pos)
    seg16 = jnp.sort(seg16, axis=-1)
    colidx = (seg16[..., None] * 16
              + jnp.arange(16, dtype=jnp.int32)).reshape(b * n_out, 256)
    cand = jnp.take_along_axis(d2, colidx, axis=1)
    _, pos256 = jax.lax.top_k(-cand, _K)
    knn_idx = jnp.take_along_axis(colidx, pos256, axis=1).reshape(
        b, n_out, _K)

    xW = _matmul(x.reshape(b * n, d_in), W_in[3:]).reshape(b, n, d_out)
    g = jax.vmap(lambda t, i: t[i])(xW, knn_idx)
    gmax = jnp.max(g, axis=-2)

    posw = fp_p_pos @ W_in[:3]
    h = _head(gmax.reshape(b * n_out, d_out),
              posw.reshape(b * n_out, d_out), W_out)
    return (h.reshape(b, n_out, d_out), fp_p_pos)


# speedup vs baseline: 13.0846x; 1.5367x over previous
import functools

import jax
import jax.numpy as jnp
from jax.experimental import pallas as pl
from jax.experimental.pallas import tpu as pltpu

_STRIDE, _K = 4, 16


def _fps_kernel(px_ref, py_ref, pz_ref, sx_ref, sy_ref, sz_ref,
                idx_ref, fx_ref, fy_ref, fz_ref,
                *, n_out, rows, nb):
    lin = (jax.lax.broadcasted_iota(jnp.int32, (rows, 128), 0) * 128
           + jax.lax.broadcasted_iota(jnp.int32, (rows, 128), 1))
    orows = n_out // 128
    lin_o = (jax.lax.broadcasted_iota(jnp.int32, (orows, 128), 0) * 128
             + jax.lax.broadcasted_iota(jnp.int32, (orows, 128), 1))
    inf = jnp.float32(jnp.inf)
    big = jnp.int32(2147483647)
    p = [(px_ref[b1], py_ref[b1], pz_ref[b1]) for b1 in range(nb)]

    def step(b1, i, dists, cur, accs):
        acc_i, acc_x, acc_y, acc_z = accs
        cx = sx_ref[b1, cur]
        cy = sy_ref[b1, cur]
        cz = sz_ref[b1, cur]
        px, py, pz = p[b1]
        dx = px - cx
        dy = py - cy
        dz = pz - cz
        d = (dx * dx + dy * dy) + dz * dz
        dists = jnp.minimum(dists, d)
        m = jnp.max(dists)
        nxt = jnp.min(jnp.where(dists == m, lin, big))
        slot_prev = (lin_o == (i - 1))
        slot_cur = (lin_o == i)
        acc_x = jnp.where(slot_prev, cx, acc_x)
        acc_y = jnp.where(slot_prev, cy, acc_y)
        acc_z = jnp.where(slot_prev, cz, acc_z)
        acc_i = jnp.where(slot_cur, nxt, acc_i)
        return dists, nxt, (acc_i, acc_x, acc_y, acc_z)

    def body(i, carry):
        out = []
        for b1 in range(nb):
            dists, cur, accs = carry[b1]
            out.append(step(b1, i, dists, cur, accs))
        return tuple(out)

    def init_one():
        return (jnp.full((rows, 128), inf, dtype=jnp.float32),
                jnp.int32(0),
                (jnp.zeros((orows, 128), dtype=jnp.int32),
                 jnp.zeros((orows, 128), dtype=jnp.float32),
                 jnp.zeros((orows, 128), dtype=jnp.float32),
                 jnp.zeros((orows, 128), dtype=jnp.float32)))

    carry = jax.lax.fori_loop(1, n_out, body,
                              tuple(init_one() for _ in range(nb)))
    slot_last = (lin_o == (n_out - 1))
    for b1 in range(nb):
        _, last, (acc_i, acc_x, acc_y, acc_z) = carry[b1]
        idx_ref[b1] = acc_i
        fx_ref[b1] = jnp.where(slot_last, sx_ref[b1, last], acc_x)
        fy_ref[b1] = jnp.where(slot_last, sy_ref[b1, last], acc_y)
        fz_ref[b1] = jnp.where(slot_last, sz_ref[b1, last], acc_z)


def _fps(p_pos, n_out):
    b, n, _ = p_pos.shape
    rows = n // 128
    orows = n_out // 128
    px = p_pos[..., 0].reshape(b, rows, 128)
    py = p_pos[..., 1].reshape(b, rows, 128)
    pz = p_pos[..., 2].reshape(b, rows, 128)
    sx = p_pos[..., 0]
    sy = p_pos[..., 1]
    sz = p_pos[..., 2]
    spec_v = pl.BlockSpec((b, rows, 128), lambda: (0, 0, 0))
    spec_s = pl.BlockSpec(memory_space=pltpu.SMEM)
    spec_out = pl.BlockSpec((b, orows, 128), lambda: (0, 0, 0))
    out_shape = [jax.ShapeDtypeStruct((b, orows, 128), jnp.int32)] + \
        [jax.ShapeDtypeStruct((b, orows, 128), jnp.float32)] * 3
    idx, fx, fy, fz = pl.pallas_call(
        functools.partial(_fps_kernel, n_out=n_out, rows=rows, nb=b),
        in_specs=[spec_v] * 3 + [spec_s] * 3,
        out_specs=[spec_out] * 4,
        out_shape=out_shape,
    )(px, py, pz, sx, sy, sz)
    fp = jnp.stack([fx.reshape(b, n_out), fy.reshape(b, n_out),
                    fz.reshape(b, n_out)], axis=-1)
    return idx.reshape(b, n_out), fp


def _knn_ab_kernel(q_ref, p_ref, d2_ref, seg_ref, m_ref, *, n, nseg):
    q8 = q_ref[...]
    q2 = jnp.sum(q8 * q8, axis=1, keepdims=True)
    ch = 1024
    for c in range(n // ch):
        sl = slice(c * ch, (c + 1) * ch)
        p8c = p_ref[0, :, sl]
        dt = jnp.dot(q8, p8c, preferred_element_type=jnp.float32)
        p2 = jnp.sum(p8c * p8c, axis=0, keepdims=True)
        d2c = (q2 - 2.0 * dt) + p2
        d2_ref[:, sl] = d2c
        m_ref[:, c * (ch // 128):(c + 1) * (ch // 128)] = jnp.min(
            d2c.reshape(128, ch // 128, 128), axis=2)

    segio = jax.lax.broadcasted_iota(jnp.int32, (128, nseg), 1)
    colio = jax.lax.broadcasted_iota(jnp.int32, (128, 16), 1)
    acc = jnp.zeros((128, 16), dtype=jnp.int32)
    big = jnp.int32(2147483647)
    inf = jnp.float32(jnp.inf)
    for t in range(16):
        M = m_ref[...]
        m = jnp.min(M, axis=1, keepdims=True)
        sid = jnp.min(jnp.where(M == m, segio, big), axis=1, keepdims=True)
        m_ref[...] = jnp.where((M == m) & (segio == sid), inf, M)
        acc = jnp.where(colio == t, sid, acc)
    seg_ref[...] = acc


def _knn_ab(fp_pos, p_pos):
    b, n_out, _ = fp_pos.shape
    n = p_pos.shape[1]
    nq = b * n_out
    nseg = n // 128
    q8 = jnp.pad(fp_pos.reshape(nq, 3), ((0, 0), (0, 5)))
    p8t = jnp.pad(jnp.swapaxes(p_pos, 1, 2), ((0, 0), (0, 5), (0, 0)))
    blocks_per_b = n_out // 128
    d2, seg16 = pl.pallas_call(
        functools.partial(_knn_ab_kernel, n=n, nseg=nseg),
        grid=(nq // 128,),
        in_specs=[
            pl.BlockSpec((128, 8), lambda i: (i, 0)),
            pl.BlockSpec((1, 8, n), lambda i: (i // blocks_per_b, 0, 0)),
        ],
        out_specs=[
            pl.BlockSpec((128, n), lambda i: (i, 0)),
            pl.BlockSpec((128, 16), lambda i: (i, 0)),
        ],
        out_shape=[
            jax.ShapeDtypeStruct((nq, n), jnp.float32),
            jax.ShapeDtypeStruct((nq, 16), jnp.int32),
        ],
        scratch_shapes=[pltpu.VMEM((128, nseg), jnp.float32)],
    )(q8, p8t)
    return d2, seg16


def _sc_gather_cand_kernel(d2_hbm, seg_hbm, cand_hbm,
                           seg_v, cand_v, sem1,
                           *, q_per_w, nc, n):
    from jax import lax
    wid = lax.axis_index("s") * nc + lax.axis_index("c")
    q0 = wid * q_per_w

    def per_query(qi, _):
        q = q0 + qi
        pltpu.sync_copy(seg_hbm.at[q], seg_v)
        seg_v[...] = seg_v[...] + q * (n // 128)
        pltpu.async_copy(d2_hbm.at[seg_v], cand_v, sem1).wait()
        pltpu.sync_copy(cand_v, cand_hbm.at[q])
        return 0

    lax.fori_loop(0, q_per_w, per_query, 0)


def _sc_gather_pool_kernel(xw_hbm, knn_hbm, out_hbm,
                           pick_v, xrows_v, gout_v, sem1,
                           *, q_per_w, nc, d_out, n_out, n):
    from jax import lax
    wid = lax.axis_index("s") * nc + lax.axis_index("c")
    q0 = wid * q_per_w

    def per_query(qi, _):
        q = q0 + qi
        pltpu.sync_copy(knn_hbm.at[q], pick_v)
        base = (q // n_out) * n
        pick_v[...] = pick_v[...] + base
        pltpu.async_copy(xw_hbm.at[pick_v], xrows_v, sem1).wait()

        def pool(c, _):
            r = xrows_v[0, pl.ds(c * 16, 16)]
            for j in range(1, 16):
                r = jnp.maximum(r, xrows_v[j, pl.ds(c * 16, 16)])
            gout_v[pl.ds(c * 16, 16)] = r
            return 0

        lax.fori_loop(0, d_out // 16, pool, 0)
        pltpu.sync_copy(gout_v, out_hbm.at[q])
        return 0

    lax.fori_loop(0, q_per_w, per_query, 0)


def _sc_meshinfo():
    from jax.experimental.pallas import tpu_sc as plsc
    info = plsc.get_sparse_core_info()
    mesh = plsc.VectorSubcoreMesh(core_axis_name="c", subcore_axis_name="s")
    return info.num_cores, info.num_subcores, mesh


def _sc_gather_cand(d2, seg16, nq):
    n = d2.shape[1]
    nc, ns, mesh = _sc_meshinfo()
    q_per_w = nq // (nc * ns)
    d2rows = d2.reshape(nq * (n // 128), 128)
    kfn = pl.kernel(
        functools.partial(_sc_gather_cand_kernel, q_per_w=q_per_w, nc=nc,
                          n=n),
        mesh=mesh,
        out_type=jax.ShapeDtypeStruct((nq, 16, 128), jnp.float32),
        scratch_types=[
            pltpu.VMEM((16,), jnp.int32),
            pltpu.VMEM((16, 128), jnp.float32),
            pltpu.SemaphoreType.DMA,
        ],
    )
    return kfn(d2rows, seg16)


def _sc_gather_pool(xw, knn, nq, d_out, n_out, n):
    nc, ns, mesh = _sc_meshinfo()
    q_per_w = nq // (nc * ns)
    kfn = pl.kernel(
        functools.partial(_sc_gather_pool_kernel, q_per_w=q_per_w, nc=nc,
                          d_out=d_out, n_out=n_out, n=n),
        mesh=mesh,
        out_type=jax.ShapeDtypeStruct((nq, d_out), jnp.float32),
        scratch_types=[
            pltpu.VMEM((16,), jnp.int32),
            pltpu.VMEM((16, d_out), jnp.float32),
            pltpu.VMEM((d_out,), jnp.float32),
            pltpu.SemaphoreType.DMA,
        ],
    )
    return kfn(xw, knn)


def _topkcand_kernel(cand_ref, cidx_ref, knn_ref, w_ref, *, width):
    ch = 512
    nch = width // ch
    w_ref[...] = cand_ref[...]
    acc = jnp.zeros((128, 16), dtype=jnp.int32)
    colio = jax.lax.broadcasted_iota(jnp.int32, (128, 16), 1)
    big = jnp.int32(2147483647)
    inf = jnp.float32(jnp.inf)
    for t in range(16):
        m = jnp.full((128, 1), inf, dtype=jnp.float32)
        for c in range(nch):
            sl = slice(c * ch, (c + 1) * ch)
            m = jnp.minimum(m, jnp.min(w_ref[:, sl], axis=1, keepdims=True))
        iw = jnp.full((128, 1), big, dtype=jnp.int32)
        for c in range(nch):
            sl = slice(c * ch, (c + 1) * ch)
            iw = jnp.minimum(iw, jnp.min(
                jnp.where(w_ref[:, sl] == m, cidx_ref[:, sl], big),
                axis=1, keepdims=True))
        for c in range(nch):
            sl = slice(c * ch, (c + 1) * ch)
            wc = w_ref[:, sl]
            w_ref[:, sl] = jnp.where((wc == m) & (cidx_ref[:, sl] == iw),
                                     inf, wc)
        acc = jnp.where(colio == t, iw, acc)
    knn_ref[...] = acc


def _topkcand(cand, cidx, nq):
    width = cand.shape[1]
    return pl.pallas_call(
        functools.partial(_topkcand_kernel, width=width),
        grid=(nq // 128,),
        in_specs=[pl.BlockSpec((128, width), lambda i: (i, 0)),
                  pl.BlockSpec((128, width), lambda i: (i, 0))],
        out_specs=pl.BlockSpec((128, 16), lambda i: (i, 0)),
        out_shape=jax.ShapeDtypeStruct((nq, 16), jnp.int32),
        scratch_shapes=[pltpu.VMEM((128, width), jnp.float32)],
    )(cand, cidx)


def _mm_kernel(a_ref, b_ref, o_ref):
    o_ref[...] = jnp.dot(a_ref[...], b_ref[...],
                         preferred_element_type=jnp.float32)


def _matmul(a, b, bm=512):
    M, Kd = a.shape
    _, N = b.shape
    return pl.pallas_call(
        _mm_kernel,
        grid=(M // bm,),
        in_specs=[pl.BlockSpec((bm, Kd), lambda i: (i, 0)),
                  pl.BlockSpec((Kd, N), lambda i: (0, 0))],
        out_specs=pl.BlockSpec((bm, N), lambda i: (i, 0)),
        out_shape=jax.ShapeDtypeStruct((M, N), jnp.float32),
    )(a, b)


def _head_kernel(g_ref, pw_ref, w_ref, o_ref):
    h = g_ref[...] + pw_ref[...]
    h = jnp.where(h > 0, h, 0.01 * h)
    o_ref[...] = jnp.dot(h, w_ref[...], preferred_element_type=jnp.float32)


def _head(gmax, posw, w_out, bm=512):
    M, D = gmax.shape
    return pl.pallas_call(
        _head_kernel,
        grid=(M // bm,),
        in_specs=[pl.BlockSpec((bm, D), lambda i: (i, 0)),
                  pl.BlockSpec((bm, D), lambda i: (i, 0)),
                  pl.BlockSpec((D, D), lambda i: (0, 0))],
        out_specs=pl.BlockSpec((bm, D), lambda i: (i, 0)),
        out_shape=jax.ShapeDtypeStruct((M, D), jnp.float32),
    )(gmax, posw, w_out)


def kernel(x, p_pos, W_in, W_out):
    b, n, d_in = x.shape
    n_out = n // _STRIDE
    d_out = W_in.shape[1]

    fps_idx, fp_p_pos = _fps(p_pos, n_out)

    d2, seg16 = _knn_ab(fp_p_pos, p_pos)
    nq = b * n_out
    cand = _sc_gather_cand(d2, seg16, nq)
    cidx = (seg16[:, :, None] * 128
            + jnp.arange(128, dtype=jnp.int32)).reshape(nq, 2048)
    knn = _topkcand(cand.reshape(nq, 2048), cidx, nq)

    xW = _matmul(x.reshape(b * n, d_in), W_in[3:])
    gmax = _sc_gather_pool(xW, knn, nq, d_out, n_out, n)

    posw = fp_p_pos @ W_in[:3]
    h = _head(gmax, posw.reshape(b * n_out, d_out), W_out)
    return (h.reshape(b, n_out, d_out), fp_p_pos)
